# baseline (speedup 1.0000x reference)
import functools

import jax
import jax.numpy as jnp
from jax import lax
from jax.experimental import pallas as pl
from jax.experimental.pallas import tpu as pltpu
from jax.experimental.pallas import tpu_sc as plsc

_N = 10000
_E = 160000
_D = 256
_H = 128
_C = 2

_NP = 10240
_NC, _NS = 2, 16
_NW = _NC * _NS
_EP = 163840
_EPW = _EP // _NW
_CH = 128
_NCHUNK = _EPW // _CH
_RPS = _NP // _NS

_mesh = plsc.VectorSubcoreMesh(core_axis_name="c", subcore_axis_name="s")


def _deg_body(col2_hbm, val_hbm, zero_hbm, out_hbm, colbuf, valbuf, degacc, dsem):
    c = lax.axis_index("c")
    s = lax.axis_index("s")
    wid = c * _NS + s
    pltpu.sync_copy(zero_hbm, degacc.at[pl.ds(s * _RPS, _RPS)])
    pltpu.sync_copy(val_hbm, valbuf)
    pltpu.sync_copy(col2_hbm.at[pl.ds(wid * _NCHUNK, _NCHUNK)], colbuf)
    plsc.subcore_barrier()

    def _fire(j, carry):
        pltpu.async_copy(valbuf, degacc.at[colbuf.at[j]], dsem, add=True)
        return carry

    lax.fori_loop(0, _NCHUNK, _fire, 0)

    def _drain(j, carry):
        pltpu.make_async_copy(valbuf, degacc.at[colbuf.at[j]], dsem).wait()
        return carry

    lax.fori_loop(0, _NCHUNK, _drain, 0)
    plsc.subcore_barrier()
    pltpu.sync_copy(degacc.at[pl.ds(s * _RPS, _RPS)],
                    out_hbm.at[c, pl.ds(s * _RPS, _RPS)])


_deg_call = functools.partial(
    pl.kernel,
    out_type=jax.ShapeDtypeStruct((_NC, _NP, _H), jnp.float32),
    mesh=_mesh,
    scratch_types=[
        pltpu.VMEM((_NCHUNK, _CH), jnp.int32),
        pltpu.VMEM((_CH, _H), jnp.float32),
        pltpu.VMEM_SHARED((_NP, _H), jnp.float32),
        pltpu.SemaphoreType.DMA,
    ],
)(_deg_body)


def _scat_body(hp_hbm, row2_hbm, col2_hbm, zero_hbm, out_hbm,
               rowbuf, colbuf, gbuf, accum, sem, sem2):
    c = lax.axis_index("c")
    s = lax.axis_index("s")
    wid = c * _NS + s
    pltpu.sync_copy(zero_hbm, accum.at[pl.ds(s * _RPS, _RPS)])
    pltpu.sync_copy(row2_hbm.at[pl.ds(wid * _NCHUNK, _NCHUNK)], rowbuf)
    pltpu.sync_copy(col2_hbm.at[pl.ds(wid * _NCHUNK, _NCHUNK)], colbuf)
    plsc.subcore_barrier()

    def _chunk(j, carry):
        h1 = pltpu.async_copy(hp_hbm.at[rowbuf.at[j, pl.ds(0, 64)]],
                              gbuf.at[pl.ds(0, 64)], sem)
        h2 = pltpu.async_copy(hp_hbm.at[rowbuf.at[j, pl.ds(64, 64)]],
                              gbuf.at[pl.ds(64, 64)], sem2)
        h1.wait()
        h2.wait()
        pltpu.sync_copy(gbuf, accum.at[colbuf.at[j]], add=True)
        return carry

    lax.fori_loop(0, _NCHUNK, _chunk, 0)
    plsc.subcore_barrier()
    pltpu.sync_copy(accum.at[pl.ds(s * _RPS, _RPS)],
                    out_hbm.at[c, pl.ds(s * _RPS, _RPS)])


_scat_call = functools.partial(
    pl.kernel,
    out_type=jax.ShapeDtypeStruct((_NC, _NP, _H), jnp.float32),
    mesh=_mesh,
    scratch_types=[
        pltpu.VMEM((_NCHUNK, _CH), jnp.int32),
        pltpu.VMEM((_NCHUNK, _CH), jnp.int32),
        pltpu.VMEM((_CH, _H), jnp.float32),
        pltpu.VMEM_SHARED((_NP, _H), jnp.float32),
        pltpu.SemaphoreType.DMA,
        pltpu.SemaphoreType.DMA,
    ],
)(_scat_body)


def _mm_body(x_ref, w1_ref, h_ref):
    h_ref[...] = jnp.dot(x_ref[...], w1_ref[...],
                         preferred_element_type=jnp.float32)


_BM = 256


def _mm_call(x_p, W1):
    grid = (_NP // _BM,)
    return pl.pallas_call(
        _mm_body,
        grid=grid,
        in_specs=[
            pl.BlockSpec((_BM, _D), lambda i: (i, 0)),
            pl.BlockSpec((_D, _H), lambda i: (0, 0)),
        ],
        out_specs=pl.BlockSpec((_BM, _H), lambda i: (i, 0)),
        out_shape=jax.ShapeDtypeStruct((_NP, _H), jnp.float32),
    )(x_p, W1)


def _deg_from_partials(degp_blk):
    d = degp_blk[0] + degp_blk[1]
    return jnp.sum(d, axis=1, keepdims=True) + 1.0


def _scale_body(h_ref, degp_ref, hp_ref):
    dis = jax.lax.rsqrt(_deg_from_partials(degp_ref[...]))
    hp_ref[...] = h_ref[...] * dis


def _scale_call(h, degp):
    grid = (_NP // _BM,)
    return pl.pallas_call(
        _scale_body,
        grid=grid,
        in_specs=[
            pl.BlockSpec((_BM, _H), lambda i: (i, 0)),
            pl.BlockSpec((_NC, _BM, _H), lambda i: (0, i, 0)),
        ],
        out_specs=pl.BlockSpec((_BM, _H), lambda i: (i, 0)),
        out_shape=jax.ShapeDtypeStruct((_NP, _H), jnp.float32),
    )(h, degp)


def _tail_body(p_ref, hp_ref, degp_ref, b1_ref, w2_ref, b2_ref, out_ref):
    dis = jax.lax.rsqrt(_deg_from_partials(degp_ref[...]))
    sums = p_ref[0] + p_ref[1] + hp_ref[...]
    pre = sums * dis + b1_ref[...]
    act = jnp.maximum(pre, 0.0)
    out_ref[...] = jnp.dot(act, w2_ref[...],
                           preferred_element_type=jnp.float32) + b2_ref[...]


def _tail_call(partials, hp, degp, b1r, W2p, b2p):
    grid = (_NP // _BM,)
    return pl.pallas_call(
        _tail_body,
        grid=grid,
        in_specs=[
            pl.BlockSpec((_NC, _BM, _H), lambda i: (0, i, 0)),
            pl.BlockSpec((_BM, _H), lambda i: (i, 0)),
            pl.BlockSpec((_NC, _BM, _H), lambda i: (0, i, 0)),
            pl.BlockSpec((1, _H), lambda i: (0, 0)),
            pl.BlockSpec((_H, 8), lambda i: (0, 0)),
            pl.BlockSpec((1, 8), lambda i: (0, 0)),
        ],
        out_specs=pl.BlockSpec((_BM, 8), lambda i: (i, 0)),
        out_shape=jax.ShapeDtypeStruct((_NP, 8), jnp.float32),
    )(partials, hp, degp, b1r, W2p, b2p)


def kernel(x, edge_index, W1, b1, W2, b2):
    row = edge_index[0]
    col = edge_index[1]
    pad = _EP - _E
    rowp = jnp.concatenate([row, jnp.zeros((pad,), jnp.int32)])
    colp = jnp.concatenate([col, jnp.full((pad,), _N, jnp.int32)])
    row2 = rowp.reshape(_EP // _CH, _CH)
    col2 = colp.reshape(_EP // _CH, _CH)
    zero_blk = jnp.zeros((_RPS, _H), jnp.float32)
    val128 = jnp.full((_CH, _H), 1.0 / _H, jnp.float32)
    b1r = b1.reshape(1, _H)
    W2p = jnp.pad(W2, ((0, 0), (0, 8 - _C)))
    b2p = jnp.pad(b2, (0, 8 - _C)).reshape(1, 8)

    degp = _deg_call(col2, val128, zero_blk)
    h = _mm_call(x, W1)
    hp = _scale_call(h, degp)
    partials = _scat_call(hp, row2, col2, zero_blk)
    out = _tail_call(partials, hp, degp, b1r, W2p, b2p)
    return out[:_N, :_C]


# speedup vs baseline: 1.1068x; 1.1068x over previous
import functools

import jax
import jax.numpy as jnp
from jax import lax
from jax.experimental import pallas as pl
from jax.experimental.pallas import tpu as pltpu
from jax.experimental.pallas import tpu_sc as plsc

_N = 10000
_E = 160000
_D = 256
_H = 128
_C = 2

_NP = 10240
_NC, _NS = 2, 16
_NW = _NC * _NS
_EP = 163840
_EPW = _EP // _NW
_CH = 128
_NCHUNK = _EPW // _CH
_RPS = _NP // _NS

_mesh = plsc.VectorSubcoreMesh(core_axis_name="c", subcore_axis_name="s")


def _deg_body(col2_hbm, val_hbm, zero_hbm, out_hbm, colbuf, valbuf, degacc, dsem):
    c = lax.axis_index("c")
    s = lax.axis_index("s")
    wid = c * _NS + s
    pltpu.sync_copy(zero_hbm, degacc.at[pl.ds(s * _RPS, _RPS)])
    pltpu.sync_copy(val_hbm, valbuf)
    pltpu.sync_copy(col2_hbm.at[pl.ds(wid * _NCHUNK, _NCHUNK)], colbuf)
    plsc.subcore_barrier()

    def _fire(j, carry):
        pltpu.async_copy(valbuf, degacc.at[colbuf.at[j]], dsem, add=True)
        return carry

    lax.fori_loop(0, _NCHUNK, _fire, 0)

    def _drain(j, carry):
        pltpu.make_async_copy(valbuf, degacc.at[colbuf.at[j]], dsem).wait()
        return carry

    lax.fori_loop(0, _NCHUNK, _drain, 0)
    plsc.subcore_barrier()
    pltpu.sync_copy(degacc.at[pl.ds(s * _RPS, _RPS)],
                    out_hbm.at[c, pl.ds(s * _RPS, _RPS)])


_deg_call = functools.partial(
    pl.kernel,
    out_type=jax.ShapeDtypeStruct((_NC, _NP, _H), jnp.float32),
    mesh=_mesh,
    scratch_types=[
        pltpu.VMEM((_NCHUNK, _CH), jnp.int32),
        pltpu.VMEM((_CH, _H), jnp.float32),
        pltpu.VMEM_SHARED((_NP, _H), jnp.float32),
        pltpu.SemaphoreType.DMA,
    ],
)(_deg_body)


def _scat_body(hp_hbm, row2_hbm, col2_hbm, zero_hbm, out_hbm,
               rowbuf, colbuf, gbuf, accum, sem, sem2):
    c = lax.axis_index("c")
    s = lax.axis_index("s")
    wid = c * _NS + s
    pltpu.sync_copy(zero_hbm, accum.at[pl.ds(s * _RPS, _RPS)])
    pltpu.sync_copy(row2_hbm.at[pl.ds(wid * _NCHUNK, _NCHUNK)], rowbuf)
    pltpu.sync_copy(col2_hbm.at[pl.ds(wid * _NCHUNK, _NCHUNK)], colbuf)
    plsc.subcore_barrier()

    def _chunk(j, carry):
        h1 = pltpu.async_copy(hp_hbm.at[rowbuf.at[j, pl.ds(0, 64)]],
                              gbuf.at[pl.ds(0, 64)], sem)
        h2 = pltpu.async_copy(hp_hbm.at[rowbuf.at[j, pl.ds(64, 64)]],
                              gbuf.at[pl.ds(64, 64)], sem2)
        h1.wait()
        h2.wait()
        pltpu.sync_copy(gbuf, accum.at[colbuf.at[j]], add=True)
        return carry

    lax.fori_loop(0, _NCHUNK, _chunk, 0)
    plsc.subcore_barrier()
    pltpu.sync_copy(accum.at[pl.ds(s * _RPS, _RPS)],
                    out_hbm.at[c, pl.ds(s * _RPS, _RPS)])


_scat_call = functools.partial(
    pl.kernel,
    out_type=jax.ShapeDtypeStruct((_NC, _NP, _H), jnp.float32),
    mesh=_mesh,
    scratch_types=[
        pltpu.VMEM((_NCHUNK, _CH), jnp.int32),
        pltpu.VMEM((_NCHUNK, _CH), jnp.int32),
        pltpu.VMEM((_CH, _H), jnp.float32),
        pltpu.VMEM_SHARED((_NP, _H), jnp.float32),
        pltpu.SemaphoreType.DMA,
        pltpu.SemaphoreType.DMA,
    ],
)(_scat_body)


def _mm_body(x_ref, w1_ref, h_ref):
    h_ref[...] = jnp.dot(x_ref[...], w1_ref[...],
                         preferred_element_type=jnp.float32)


_BM = 256


def _mm_call(x_p, W1):
    grid = (_NP // _BM,)
    return pl.pallas_call(
        _mm_body,
        grid=grid,
        in_specs=[
            pl.BlockSpec((_BM, _D), lambda i: (i, 0)),
            pl.BlockSpec((_D, _H), lambda i: (0, 0)),
        ],
        out_specs=pl.BlockSpec((_BM, _H), lambda i: (i, 0)),
        out_shape=jax.ShapeDtypeStruct((_NP, _H), jnp.float32),
    )(x_p, W1)


def _deg_from_partials(degp_blk):
    d = degp_blk[0] + degp_blk[1]
    return jnp.sum(d, axis=1, keepdims=True) + 1.0


def _scale_body(h_ref, degp_ref, hp_ref):
    dis = jax.lax.rsqrt(_deg_from_partials(degp_ref[...]))
    hp_ref[...] = h_ref[...] * dis


def _scale_call(h, degp):
    grid = (_NP // _BM,)
    return pl.pallas_call(
        _scale_body,
        grid=grid,
        in_specs=[
            pl.BlockSpec((_BM, _H), lambda i: (i, 0)),
            pl.BlockSpec((_NC, _BM, _H), lambda i: (0, i, 0)),
        ],
        out_specs=pl.BlockSpec((_BM, _H), lambda i: (i, 0)),
        out_shape=jax.ShapeDtypeStruct((_NP, _H), jnp.float32),
    )(h, degp)


def _tail_body(p_ref, hp_ref, degp_ref, b1_ref, w2_ref, b2_ref, out_ref):
    dis = jax.lax.rsqrt(_deg_from_partials(degp_ref[...]))
    sums = p_ref[0] + p_ref[1] + hp_ref[...]
    pre = sums * dis + b1_ref[...]
    act = jnp.maximum(pre, 0.0)
    out_ref[...] = jnp.dot(act, w2_ref[...],
                           preferred_element_type=jnp.float32) + b2_ref[...]


def _tail_call(partials, hp, degp, b1r, W2p, b2p):
    grid = (_NP // _BM,)
    return pl.pallas_call(
        _tail_body,
        grid=grid,
        in_specs=[
            pl.BlockSpec((_NC, _BM, _H), lambda i: (0, i, 0)),
            pl.BlockSpec((_BM, _H), lambda i: (i, 0)),
            pl.BlockSpec((_NC, _BM, _H), lambda i: (0, i, 0)),
            pl.BlockSpec((1, _H), lambda i: (0, 0)),
            pl.BlockSpec((_H, 8), lambda i: (0, 0)),
            pl.BlockSpec((1, 8), lambda i: (0, 0)),
        ],
        out_specs=pl.BlockSpec((_BM, 8), lambda i: (i, 0)),
        out_shape=jax.ShapeDtypeStruct((_NP, 8), jnp.float32),
    )(partials, hp, degp, b1r, W2p, b2p)


def kernel(x, edge_index, W1, b1, W2, b2):
    row = edge_index[0]
    col = edge_index[1]
    pad = _EP - _E
    rowp = jnp.concatenate([row, jnp.zeros((pad,), jnp.int32)])
    colp = jnp.concatenate([col, jnp.full((pad,), _N, jnp.int32)])
    row2 = rowp.reshape(_EP // _CH, _CH)
    col2 = colp.reshape(_EP // _CH, _CH)
    x_p = jnp.concatenate([x, jnp.zeros((_NP - _N, _D), jnp.float32)])
    zero_blk = jnp.zeros((_RPS, _H), jnp.float32)
    val128 = jnp.full((_CH, _H), 1.0 / _H, jnp.float32)
    b1r = b1.reshape(1, _H)
    W2p = jnp.pad(W2, ((0, 0), (0, 8 - _C)))
    b2p = jnp.pad(b2, (0, 8 - _C)).reshape(1, 8)

    degp = _deg_call(col2, val128, zero_blk)
    h = _mm_call(x_p, W1)
    hp = _scale_call(h, degp)
    partials = _scat_call(hp, row2, col2, zero_blk)
    out = _tail_call(partials, hp, degp, b1r, W2p, b2p)
    return out[:_N, :_C]
